# baseline (device time: 14752 ns/iter reference)
import jax
import jax.numpy as jnp
from jax import lax
from jax.experimental import pallas as pl
from jax.experimental.pallas import tpu as pltpu

N_DEV = 4
B, SQ_PER, SKV_PER, HQ, DH = 2, 128, 128, 4, 64
D_MODEL = 512
D_QK = HQ * DH
BLK = 64


def kernel(x, Wq, K_ext, V_ext, Wo):
    def body(x_hbm, wq_hbm, kt_hbm, vt_hbm, wo_hbm, out_hbm,
             xs, wqs, kts, vts, wos, os_ref, kv_ref,
             in_sems, out_sems, send_sems, recv_sems):
        my = lax.axis_index("i")
        partner = (my + 2) % N_DEV

        barrier_sem = pltpu.get_barrier_semaphore()
        pl.semaphore_signal(
            barrier_sem, inc=1,
            device_id=(partner,), device_id_type=pl.DeviceIdType.MESH,
        )

        dmas = []
        for i, (src, dst) in enumerate(
                [(kt_hbm, kts), (vt_hbm, vts), (x_hbm, xs),
                 (wq_hbm, wqs), (wo_hbm, wos)]):
            d = pltpu.make_async_copy(src, dst, in_sems.at[i])
            d.start()
            dmas.append(d)

        def send(b):
            r = pltpu.make_async_remote_copy(
                src_ref=kv_ref.at[0, b], dst_ref=kv_ref.at[1, b],
                send_sem=send_sems.at[b], recv_sem=recv_sems.at[b],
                device_id=(partner,), device_id_type=pl.DeviceIdType.MESH,
            )
            r.start()
            return r

        dmas[0].wait()
        dmas[1].wait()
        rdmas = []
        for b in range(B):
            kv_ref[0, b, :D_QK, :] = (
                kts[b].astype(jnp.bfloat16).reshape(D_QK, SKV_PER))
            kv_ref[0, b, D_QK:, :] = (
                vts[b].astype(jnp.bfloat16).reshape(D_QK, SKV_PER))
            if b == 0:
                pl.semaphore_wait(barrier_sem, 1)
            rdmas.append(send(b))

        dmas[2].wait()
        dmas[3].wait()
        wq = wqs[...].astype(jnp.bfloat16)
        xx = xs[...].astype(jnp.bfloat16).reshape(B * SQ_PER, D_MODEL)
        q2 = (jnp.dot(xx, wq, preferred_element_type=jnp.float32)
              * 0.125).astype(jnp.bfloat16)
        dmas[4].wait()
        wo = wos[...].astype(jnp.bfloat16)

        out_dmas = []
        for b in range(B):
            rdmas[b].wait_recv()
            ctx_rows = []
            for t in range(2):
                r0 = b * SQ_PER + t * BLK
                heads = []
                for h in range(HQ):
                    q = q2[r0:r0 + BLK, h * DH:(h + 1) * DH]
                    krows = pl.ds(h * DH, DH)
                    vrows = pl.ds(D_QK + h * DH, DH)
                    scols = pl.ds(t * BLK, BLK)
                    s_l = jnp.dot(q, kv_ref[0, b, krows, scols],
                                  preferred_element_type=jnp.float32)
                    s_r = jnp.dot(q, kv_ref[1, b, krows, scols],
                                  preferred_element_type=jnp.float32)
                    w_l = jnp.exp(s_l)
                    w_r = jnp.exp(s_r)
                    wsum = (jnp.sum(w_l, axis=-1, keepdims=True)
                            + jnp.sum(w_r, axis=-1, keepdims=True))
                    dn = (((1,), (1,)), ((), ()))
                    ctx = (
                        lax.dot_general(
                            w_l.astype(jnp.bfloat16),
                            kv_ref[0, b, vrows, scols],
                            dimension_numbers=dn,
                            preferred_element_type=jnp.float32)
                        + lax.dot_general(
                            w_r.astype(jnp.bfloat16),
                            kv_ref[1, b, vrows, scols],
                            dimension_numbers=dn,
                            preferred_element_type=jnp.float32)
                    )
                    heads.append((ctx * (1.0 / wsum)).astype(jnp.bfloat16))
                ctx_rows.append(jnp.concatenate(heads, axis=1))
            ctx_b = jnp.concatenate(ctx_rows, axis=0)
            os_ref[b] = jnp.dot(ctx_b, wo, preferred_element_type=jnp.float32)
            d = pltpu.make_async_copy(os_ref.at[b], out_hbm.at[b],
                                      out_sems.at[b])
            d.start()
            out_dmas.append(d)

        for d in out_dmas:
            d.wait()
        for b in range(B):
            rdmas[b].wait_send()

    K_t = jnp.transpose(K_ext, (0, 2, 3, 1))
    V_t = jnp.transpose(V_ext, (0, 2, 3, 1))

    return pl.pallas_call(
        body,
        out_shape=jax.ShapeDtypeStruct((B, SQ_PER, D_MODEL), jnp.float32),
        in_specs=[pl.BlockSpec(memory_space=pltpu.MemorySpace.HBM)] * 5,
        out_specs=pl.BlockSpec(memory_space=pltpu.MemorySpace.HBM),
        scratch_shapes=[
            pltpu.VMEM((B, SQ_PER, D_MODEL), jnp.float32),
            pltpu.VMEM((D_MODEL, D_QK), jnp.float32),
            pltpu.VMEM((B, HQ, DH, SKV_PER), jnp.float32),
            pltpu.VMEM((B, HQ, DH, SKV_PER), jnp.float32),
            pltpu.VMEM((D_QK, D_MODEL), jnp.float32),
            pltpu.VMEM((B, SQ_PER, D_MODEL), jnp.float32),
            pltpu.VMEM((2, B, 2 * D_QK, SKV_PER), jnp.bfloat16),
            pltpu.SemaphoreType.DMA((5,)),
            pltpu.SemaphoreType.DMA((B,)),
            pltpu.SemaphoreType.DMA((B,)),
            pltpu.SemaphoreType.DMA((B,)),
        ],
        compiler_params=pltpu.CompilerParams(collective_id=0),
    )(x, Wq, K_t, V_t, Wo)


# device time: 12666 ns/iter; 1.1647x vs baseline; 1.1647x over previous
import jax
import jax.numpy as jnp
from jax import lax
from jax.experimental import pallas as pl
from jax.experimental.pallas import tpu as pltpu

N_DEV = 4
B, SQ_PER, SKV_PER, HQ, DH = 2, 128, 128, 4, 64
D_MODEL = 512
D_QK = HQ * DH
BLK = 64


def kernel(x, Wq, K_ext, V_ext, Wo):
    def body(x_ref, wq_ref, kt_ref, vt_ref, wo_ref, out_ref,
             rk_ref, rv_ref, send_sems, recv_sems):
        my = lax.axis_index("i")
        partner = (my + 2) % N_DEV

        barrier_sem = pltpu.get_barrier_semaphore()
        pl.semaphore_signal(
            barrier_sem, inc=1,
            device_id=(partner,), device_id_type=pl.DeviceIdType.MESH,
        )
        pl.semaphore_wait(barrier_sem, 1)

        def send(src, dst, i):
            r = pltpu.make_async_remote_copy(
                src_ref=src, dst_ref=dst,
                send_sem=send_sems.at[i], recv_sem=recv_sems.at[i],
                device_id=(partner,), device_id_type=pl.DeviceIdType.MESH,
            )
            r.start()
            return r

        rdmas = []
        for b in range(B):
            rdmas.append(send(kt_ref.at[b], rk_ref.at[b], 2 * b))
            rdmas.append(send(vt_ref.at[b], rv_ref.at[b], 2 * b + 1))

        q2 = (jnp.dot(x_ref[...].reshape(B * SQ_PER, D_MODEL), wq_ref[...],
                      preferred_element_type=jnp.float32)
              ).astype(jnp.bfloat16)

        ctx_rows = []
        for b in range(B):
            rdmas[2 * b].wait_recv()
            rdmas[2 * b + 1].wait_recv()
            for t in range(2):
                r0 = b * SQ_PER + t * BLK
                heads = []
                for h in range(HQ):
                    q = q2[r0:r0 + BLK, h * DH:(h + 1) * DH]
                    sc = pl.ds(t * BLK, BLK)
                    s_l = jnp.dot(q, kt_ref[b, h, :, sc],
                                  preferred_element_type=jnp.float32)
                    s_r = jnp.dot(q, rk_ref[b, h, :, sc],
                                  preferred_element_type=jnp.float32)
                    w_l = jnp.exp(s_l)
                    w_r = jnp.exp(s_r)
                    wsum = (jnp.sum(w_l, axis=-1, keepdims=True)
                            + jnp.sum(w_r, axis=-1, keepdims=True))
                    dn = (((1,), (1,)), ((), ()))
                    ctx = (
                        lax.dot_general(
                            w_l.astype(jnp.bfloat16), vt_ref[b, h, :, sc],
                            dimension_numbers=dn,
                            preferred_element_type=jnp.float32)
                        + lax.dot_general(
                            w_r.astype(jnp.bfloat16), rv_ref[b, h, :, sc],
                            dimension_numbers=dn,
                            preferred_element_type=jnp.float32)
                    )
                    heads.append((ctx * (1.0 / wsum)).astype(jnp.bfloat16))
                ctx_rows.append(jnp.concatenate(heads, axis=1))
        ctx_all = jnp.concatenate(ctx_rows, axis=0)

        out = jnp.dot(ctx_all, wo_ref[...],
                      preferred_element_type=jnp.float32)
        out_ref[...] = out.astype(jnp.bfloat16).reshape(B, SQ_PER, D_MODEL)

        for r in rdmas:
            r.wait_send()

    xb = x.astype(jnp.bfloat16)
    wqb = (Wq * 0.125).astype(jnp.bfloat16)
    wob = Wo.astype(jnp.bfloat16)
    ktb = jnp.transpose(K_ext, (0, 2, 3, 1)).astype(jnp.bfloat16)
    vtb = jnp.transpose(V_ext, (0, 2, 3, 1)).astype(jnp.bfloat16)

    return pl.pallas_call(
        body,
        out_shape=jax.ShapeDtypeStruct((B, SQ_PER, D_MODEL), jnp.bfloat16),
        in_specs=[pl.BlockSpec(memory_space=pltpu.VMEM)] * 5,
        out_specs=pl.BlockSpec(memory_space=pltpu.VMEM),
        scratch_shapes=[
            pltpu.VMEM((B, HQ, DH, SKV_PER), jnp.bfloat16),
            pltpu.VMEM((B, HQ, DH, SKV_PER), jnp.bfloat16),
            pltpu.SemaphoreType.DMA((2 * B,)),
            pltpu.SemaphoreType.DMA((2 * B,)),
        ],
        compiler_params=pltpu.CompilerParams(collective_id=0),
    )(xb, wqb, ktb, vtb, wob)


# device time: 10571 ns/iter; 1.3955x vs baseline; 1.1982x over previous
import jax
import jax.numpy as jnp
from jax import lax
from jax.experimental import pallas as pl
from jax.experimental.pallas import tpu as pltpu

N_DEV = 4
B, SQ_PER, SKV_PER, HQ, DH = 2, 128, 128, 4, 64
D_MODEL = 512
D_QK = HQ * DH
BLK = 64


def kernel(x, Wq, K_ext, V_ext, Wo):
    def body(x_hbm, wq_hbm, kt_hbm, vt_hbm, wo_hbm, out_hbm,
             xs, wqs, kts, vts, wos, os_ref, lkv, rkv,
             in_sems, out_sems, send_sems, recv_sems):
        my = lax.axis_index("i")
        partner = (my + 2) % N_DEV

        barrier_sem = pltpu.get_barrier_semaphore()
        pl.semaphore_signal(
            barrier_sem, inc=1,
            device_id=(partner,), device_id_type=pl.DeviceIdType.MESH,
        )

        dmas = []
        for i, (src, dst) in enumerate(
                [(kt_hbm, kts), (vt_hbm, vts), (x_hbm, xs),
                 (wq_hbm, wqs), (wo_hbm, wos)]):
            d = pltpu.make_async_copy(src, dst, in_sems.at[i])
            d.start()
            dmas.append(d)

        def send(b):
            r = pltpu.make_async_remote_copy(
                src_ref=lkv.at[b], dst_ref=rkv.at[b],
                send_sem=send_sems.at[b], recv_sem=recv_sems.at[b],
                device_id=(partner,), device_id_type=pl.DeviceIdType.MESH,
            )
            r.start()
            return r

        dmas[0].wait()
        dmas[1].wait()
        rdmas = []
        for b in range(B):
            lkv[b, :D_QK, :] = (
                kts[b].astype(jnp.bfloat16).reshape(D_QK, SKV_PER))
            lkv[b, D_QK:, :] = (
                vts[b].astype(jnp.bfloat16).reshape(D_QK, SKV_PER))
            if b == 0:
                pl.semaphore_wait(barrier_sem, 1)
            rdmas.append(send(b))

        dmas[2].wait()
        dmas[3].wait()
        wq = wqs[...].astype(jnp.bfloat16)
        xx = xs[...].astype(jnp.bfloat16).reshape(B * SQ_PER, D_MODEL)
        q2 = (jnp.dot(xx, wq, preferred_element_type=jnp.float32)
              * 0.125).astype(jnp.bfloat16)
        dmas[4].wait()
        wo = wos[...].astype(jnp.bfloat16)

        out_dmas = []
        for b in range(B):
            rdmas[b].wait_recv()
            ctx_rows = []
            for t in range(2):
                r0 = b * SQ_PER + t * BLK
                heads = []
                for h in range(HQ):
                    q = q2[r0:r0 + BLK, h * DH:(h + 1) * DH]
                    kr = pl.ds(h * DH, DH)
                    vr = pl.ds(D_QK + h * DH, DH)
                    sc = pl.ds(t * BLK, BLK)
                    s_l = jnp.dot(q, lkv[b, kr, sc],
                                  preferred_element_type=jnp.float32)
                    s_r = jnp.dot(q, rkv[b, kr, sc],
                                  preferred_element_type=jnp.float32)
                    w_l = jnp.exp(s_l)
                    w_r = jnp.exp(s_r)
                    wsum = (jnp.sum(w_l, axis=-1, keepdims=True)
                            + jnp.sum(w_r, axis=-1, keepdims=True))
                    dn = (((1,), (1,)), ((), ()))
                    ctx = (
                        lax.dot_general(
                            w_l.astype(jnp.bfloat16), lkv[b, vr, sc],
                            dimension_numbers=dn,
                            preferred_element_type=jnp.float32)
                        + lax.dot_general(
                            w_r.astype(jnp.bfloat16), rkv[b, vr, sc],
                            dimension_numbers=dn,
                            preferred_element_type=jnp.float32)
                    )
                    heads.append((ctx * (1.0 / wsum)).astype(jnp.bfloat16))
                ctx_rows.append(jnp.concatenate(heads, axis=1))
            ctx_b = jnp.concatenate(ctx_rows, axis=0)
            os_ref[b] = jnp.dot(
                ctx_b, wo, preferred_element_type=jnp.float32
            ).astype(jnp.bfloat16)
            d = pltpu.make_async_copy(os_ref.at[b], out_hbm.at[b],
                                      out_sems.at[b])
            d.start()
            out_dmas.append(d)

        for d in out_dmas:
            d.wait()
        for b in range(B):
            rdmas[b].wait_send()

    K_t = jnp.transpose(K_ext, (0, 2, 3, 1))
    V_t = jnp.transpose(V_ext, (0, 2, 3, 1))

    hbm = pltpu.MemorySpace.HBM
    args = [pltpu.with_memory_space_constraint(a, hbm)
            for a in (x, Wq, K_t, V_t, Wo)]

    return pl.pallas_call(
        body,
        out_shape=jax.ShapeDtypeStruct((B, SQ_PER, D_MODEL), jnp.bfloat16),
        in_specs=[pl.BlockSpec(memory_space=hbm)] * 5,
        out_specs=pl.BlockSpec(memory_space=hbm),
        scratch_shapes=[
            pltpu.VMEM((B, SQ_PER, D_MODEL), jnp.float32),
            pltpu.VMEM((D_MODEL, D_QK), jnp.float32),
            pltpu.VMEM((B, HQ, DH, SKV_PER), jnp.float32),
            pltpu.VMEM((B, HQ, DH, SKV_PER), jnp.float32),
            pltpu.VMEM((D_QK, D_MODEL), jnp.float32),
            pltpu.VMEM((B, SQ_PER, D_MODEL), jnp.bfloat16),
            pltpu.VMEM((B, 2 * D_QK, SKV_PER), jnp.bfloat16),
            pltpu.VMEM((B, 2 * D_QK, SKV_PER), jnp.bfloat16),
            pltpu.SemaphoreType.DMA((5,)),
            pltpu.SemaphoreType.DMA((B,)),
            pltpu.SemaphoreType.DMA((B,)),
            pltpu.SemaphoreType.DMA((B,)),
        ],
        compiler_params=pltpu.CompilerParams(collective_id=0),
    )(*args)


# device time: 10480 ns/iter; 1.4076x vs baseline; 1.0087x over previous
import jax
import jax.numpy as jnp
from jax import lax
from jax.experimental import pallas as pl
from jax.experimental.pallas import tpu as pltpu

N_DEV = 4
B, SQ_PER, SKV_PER, HQ, DH = 2, 128, 128, 4, 64
D_MODEL = 512
D_QK = HQ * DH
BLK = 64


def kernel(x, Wq, K_ext, V_ext, Wo):
    def body(x_hbm, wq_hbm, kt_hbm, vt_hbm, wo_hbm, out_hbm,
             xs, wqs, kts, vts, wos, os_ref, lkv, rkv,
             in_sems, out_sems, send_sems, recv_sems):
        my = lax.axis_index("i")
        partner = (my + 2) % N_DEV

        barrier_sem = pltpu.get_barrier_semaphore()
        pl.semaphore_signal(
            barrier_sem, inc=1,
            device_id=(partner,), device_id_type=pl.DeviceIdType.MESH,
        )

        dmas = []
        for i, (src, dst) in enumerate(
                [(kt_hbm, kts), (vt_hbm, vts), (x_hbm, xs),
                 (wq_hbm, wqs), (wo_hbm, wos)]):
            d = pltpu.make_async_copy(src, dst, in_sems.at[i])
            d.start()
            dmas.append(d)

        def send(b):
            r = pltpu.make_async_remote_copy(
                src_ref=lkv.at[b], dst_ref=rkv.at[b],
                send_sem=send_sems.at[b], recv_sem=recv_sems.at[b],
                device_id=(partner,), device_id_type=pl.DeviceIdType.MESH,
            )
            r.start()
            return r

        dmas[0].wait()
        dmas[1].wait()
        rdmas = []
        for b in range(B):
            lkv[b, :D_QK, :] = (
                kts[b].astype(jnp.bfloat16).reshape(D_QK, SKV_PER))
            lkv[b, D_QK:, :] = (
                vts[b].astype(jnp.bfloat16).reshape(D_QK, SKV_PER))
            if b == 0:
                pl.semaphore_wait(barrier_sem, 1)
            rdmas.append(send(b))

        dmas[2].wait()
        dmas[3].wait()
        wq = wqs[...].astype(jnp.bfloat16)
        xx = xs[...].astype(jnp.bfloat16).reshape(B * SQ_PER, D_MODEL)
        q2 = (jnp.dot(xx, wq, preferred_element_type=jnp.float32)
              * 0.125).astype(jnp.bfloat16)
        dmas[4].wait()
        wo = wos[...].astype(jnp.bfloat16)

        dn = (((1,), (1,)), ((), ()))

        ctx_l = {}
        sum_l = {}
        for b in range(B):
            for t in range(2):
                r0 = b * SQ_PER + t * BLK
                for h in range(HQ):
                    q = q2[r0:r0 + BLK, h * DH:(h + 1) * DH]
                    kr = pl.ds(h * DH, DH)
                    vr = pl.ds(D_QK + h * DH, DH)
                    sc = pl.ds(t * BLK, BLK)
                    w_l = jnp.exp(jnp.dot(
                        q, lkv[b, kr, sc],
                        preferred_element_type=jnp.float32))
                    sum_l[b, t, h] = jnp.sum(w_l, axis=-1, keepdims=True)
                    ctx_l[b, t, h] = lax.dot_general(
                        w_l.astype(jnp.bfloat16), lkv[b, vr, sc],
                        dimension_numbers=dn,
                        preferred_element_type=jnp.float32)

        out_dmas = []
        for b in range(B):
            rdmas[b].wait_recv()
            ctx_rows = []
            for t in range(2):
                r0 = b * SQ_PER + t * BLK
                heads = []
                for h in range(HQ):
                    q = q2[r0:r0 + BLK, h * DH:(h + 1) * DH]
                    kr = pl.ds(h * DH, DH)
                    vr = pl.ds(D_QK + h * DH, DH)
                    sc = pl.ds(t * BLK, BLK)
                    w_r = jnp.exp(jnp.dot(
                        q, rkv[b, kr, sc],
                        preferred_element_type=jnp.float32))
                    wsum = sum_l[b, t, h] + jnp.sum(
                        w_r, axis=-1, keepdims=True)
                    ctx = ctx_l[b, t, h] + lax.dot_general(
                        w_r.astype(jnp.bfloat16), rkv[b, vr, sc],
                        dimension_numbers=dn,
                        preferred_element_type=jnp.float32)
                    heads.append((ctx * (1.0 / wsum)).astype(jnp.bfloat16))
                ctx_rows.append(jnp.concatenate(heads, axis=1))
            ctx_b = jnp.concatenate(ctx_rows, axis=0)
            os_ref[b] = jnp.dot(
                ctx_b, wo, preferred_element_type=jnp.float32
            ).astype(jnp.bfloat16)
            d = pltpu.make_async_copy(os_ref.at[b], out_hbm.at[b],
                                      out_sems.at[b])
            d.start()
            out_dmas.append(d)

        for d in out_dmas:
            d.wait()
        for b in range(B):
            rdmas[b].wait_send()

    K_t = jnp.transpose(K_ext, (0, 2, 3, 1))
    V_t = jnp.transpose(V_ext, (0, 2, 3, 1))

    hbm = pltpu.MemorySpace.HBM
    args = [pltpu.with_memory_space_constraint(a, hbm)
            for a in (x, Wq, K_t, V_t, Wo)]

    return pl.pallas_call(
        body,
        out_shape=jax.ShapeDtypeStruct((B, SQ_PER, D_MODEL), jnp.bfloat16),
        in_specs=[pl.BlockSpec(memory_space=hbm)] * 5,
        out_specs=pl.BlockSpec(memory_space=hbm),
        scratch_shapes=[
            pltpu.VMEM((B, SQ_PER, D_MODEL), jnp.float32),
            pltpu.VMEM((D_MODEL, D_QK), jnp.float32),
            pltpu.VMEM((B, HQ, DH, SKV_PER), jnp.float32),
            pltpu.VMEM((B, HQ, DH, SKV_PER), jnp.float32),
            pltpu.VMEM((D_QK, D_MODEL), jnp.float32),
            pltpu.VMEM((B, SQ_PER, D_MODEL), jnp.bfloat16),
            pltpu.VMEM((B, 2 * D_QK, SKV_PER), jnp.bfloat16),
            pltpu.VMEM((B, 2 * D_QK, SKV_PER), jnp.bfloat16),
            pltpu.SemaphoreType.DMA((5,)),
            pltpu.SemaphoreType.DMA((B,)),
            pltpu.SemaphoreType.DMA((B,)),
            pltpu.SemaphoreType.DMA((B,)),
        ],
        compiler_params=pltpu.CompilerParams(collective_id=0),
    )(*args)


# device time: 10025 ns/iter; 1.4715x vs baseline; 1.0454x over previous
import jax
import jax.numpy as jnp
from jax import lax
from jax.experimental import pallas as pl
from jax.experimental.pallas import tpu as pltpu

N_DEV = 4
B, SQ_PER, SKV_PER, HQ, DH = 2, 128, 128, 4, 64
D_MODEL = 512
D_QK = HQ * DH
BLK = 64


def kernel(x, Wq, K_ext, V_ext, Wo):
    def body(x_hbm, wq_hbm, kt_hbm, vt_hbm, wo_hbm, out_hbm,
             xs, wqs, kts, vts, wos, os_ref, lkv, rkv,
             in_sems, out_sems, send_sems, recv_sems):
        my = lax.axis_index("i")
        partner = (my + 2) % N_DEV

        barrier_sem = pltpu.get_barrier_semaphore()
        pl.semaphore_signal(
            barrier_sem, inc=1,
            device_id=(partner,), device_id_type=pl.DeviceIdType.MESH,
        )

        dmas = []
        for i, (src, dst) in enumerate(
                [(kt_hbm, kts), (vt_hbm, vts), (x_hbm, xs),
                 (wq_hbm, wqs), (wo_hbm, wos)]):
            d = pltpu.make_async_copy(src, dst, in_sems.at[i])
            d.start()
            dmas.append(d)

        def send(b, part, i):
            rows = pl.ds(part * D_QK, D_QK)
            r = pltpu.make_async_remote_copy(
                src_ref=lkv.at[b, rows], dst_ref=rkv.at[b, rows],
                send_sem=send_sems.at[i], recv_sem=recv_sems.at[i],
                device_id=(partner,), device_id_type=pl.DeviceIdType.MESH,
            )
            r.start()
            return r

        dmas[0].wait()
        dmas[1].wait()
        rdmas = []
        for b in range(B):
            lkv[b, :D_QK, :] = (
                kts[b].astype(jnp.bfloat16).reshape(D_QK, SKV_PER))
            lkv[b, D_QK:, :] = (
                vts[b].astype(jnp.bfloat16).reshape(D_QK, SKV_PER))
            if b == 0:
                pl.semaphore_wait(barrier_sem, 1)
            rdmas.append(send(b, 0, 2 * b))
            rdmas.append(send(b, 1, 2 * b + 1))

        dmas[2].wait()
        dmas[3].wait()
        wq = wqs[...].astype(jnp.bfloat16)
        xx = xs[...].astype(jnp.bfloat16).reshape(B * SQ_PER, D_MODEL)
        q2 = (jnp.dot(xx, wq, preferred_element_type=jnp.float32)
              * 0.125).astype(jnp.bfloat16)
        dmas[4].wait()
        wo = wos[...].astype(jnp.bfloat16)

        dn = (((1,), (1,)), ((), ()))

        ctx_l = {}
        sum_l = {}
        for b in range(B):
            for t in range(2):
                r0 = b * SQ_PER + t * BLK
                for h in range(HQ):
                    q = q2[r0:r0 + BLK, h * DH:(h + 1) * DH]
                    kr = pl.ds(h * DH, DH)
                    vr = pl.ds(D_QK + h * DH, DH)
                    sc = pl.ds(t * BLK, BLK)
                    w_l = jnp.exp(jnp.dot(
                        q, lkv[b, kr, sc],
                        preferred_element_type=jnp.float32))
                    sum_l[b, t, h] = jnp.sum(w_l, axis=-1, keepdims=True)
                    ctx_l[b, t, h] = lax.dot_general(
                        w_l.astype(jnp.bfloat16), lkv[b, vr, sc],
                        dimension_numbers=dn,
                        preferred_element_type=jnp.float32)

        out_dmas = []
        for b in range(B):
            rdmas[2 * b].wait_recv()
            w_rs = {}
            for t in range(2):
                r0 = b * SQ_PER + t * BLK
                for h in range(HQ):
                    q = q2[r0:r0 + BLK, h * DH:(h + 1) * DH]
                    w_rs[t, h] = jnp.exp(jnp.dot(
                        q, rkv[b, pl.ds(h * DH, DH), pl.ds(t * BLK, BLK)],
                        preferred_element_type=jnp.float32))
            rdmas[2 * b + 1].wait_recv()
            ctx_rows = []
            for t in range(2):
                heads = []
                for h in range(HQ):
                    w_r = w_rs[t, h]
                    vr = pl.ds(D_QK + h * DH, DH)
                    sc = pl.ds(t * BLK, BLK)
                    wsum = sum_l[b, t, h] + jnp.sum(
                        w_r, axis=-1, keepdims=True)
                    ctx = ctx_l[b, t, h] + lax.dot_general(
                        w_r.astype(jnp.bfloat16), rkv[b, vr, sc],
                        dimension_numbers=dn,
                        preferred_element_type=jnp.float32)
                    heads.append((ctx * (1.0 / wsum)).astype(jnp.bfloat16))
                ctx_rows.append(jnp.concatenate(heads, axis=1))
            ctx_b = jnp.concatenate(ctx_rows, axis=0)
            os_ref[b] = jnp.dot(
                ctx_b, wo, preferred_element_type=jnp.float32
            ).astype(jnp.bfloat16)
            d = pltpu.make_async_copy(os_ref.at[b], out_hbm.at[b],
                                      out_sems.at[b])
            d.start()
            out_dmas.append(d)

        for d in out_dmas:
            d.wait()
        for r in rdmas:
            r.wait_send()

    K_t = jnp.transpose(K_ext, (0, 2, 3, 1))
    V_t = jnp.transpose(V_ext, (0, 2, 3, 1))

    hbm = pltpu.MemorySpace.HBM
    args = [pltpu.with_memory_space_constraint(a, hbm)
            for a in (x, Wq, K_t, V_t, Wo)]

    return pl.pallas_call(
        body,
        out_shape=jax.ShapeDtypeStruct((B, SQ_PER, D_MODEL), jnp.bfloat16),
        in_specs=[pl.BlockSpec(memory_space=hbm)] * 5,
        out_specs=pl.BlockSpec(memory_space=hbm),
        scratch_shapes=[
            pltpu.VMEM((B, SQ_PER, D_MODEL), jnp.float32),
            pltpu.VMEM((D_MODEL, D_QK), jnp.float32),
            pltpu.VMEM((B, HQ, DH, SKV_PER), jnp.float32),
            pltpu.VMEM((B, HQ, DH, SKV_PER), jnp.float32),
            pltpu.VMEM((D_QK, D_MODEL), jnp.float32),
            pltpu.VMEM((B, SQ_PER, D_MODEL), jnp.bfloat16),
            pltpu.VMEM((B, 2 * D_QK, SKV_PER), jnp.bfloat16),
            pltpu.VMEM((B, 2 * D_QK, SKV_PER), jnp.bfloat16),
            pltpu.SemaphoreType.DMA((5,)),
            pltpu.SemaphoreType.DMA((B,)),
            pltpu.SemaphoreType.DMA((2 * B,)),
            pltpu.SemaphoreType.DMA((2 * B,)),
        ],
        compiler_params=pltpu.CompilerParams(collective_id=0),
    )(*args)
